# Initial kernel scaffold; baseline (speedup 1.0000x reference)
import jax
import jax.numpy as jnp
from jax.experimental import pallas as pl


def kernel(attn_weights, hidden_states, metric):
    raise NotImplementedError("write your pallas kernel here")


# speedup vs baseline: 2.0432x; 2.0432x over previous
import functools

import jax
import jax.numpy as jnp
from jax import lax
from jax.experimental import pallas as pl

B, H, S, D, DM = 8, 16, 577, 1024, 64
DOM = 54
CTX = 10
KEEP = S - (DOM + 1)
STEP = max(1, KEEP // CTX)
NSEL = DOM + 1
OUT_T = NSEL + CTX

_HIGH = lax.Precision.HIGHEST


def _tn(x, ident):
    return lax.dot_general(x, ident, (((0,), (0,)), ((), ())),
                           precision=_HIGH)


def _kernel(attn_cls_ref, hid_ref, met_ref, out_ref, idx_ref):
    f32 = jnp.float32
    attn = attn_cls_ref[0]
    v = jnp.sum(attn, axis=0, keepdims=True)
    col0 = lax.broadcasted_iota(jnp.int32, (1, S), 1) == 0
    v = jnp.where(col0, -jnp.inf, v)

    ident_s = (lax.broadcasted_iota(jnp.int32, (S, S), 0)
               == lax.broadcasted_iota(jnp.int32, (S, S), 1)).astype(f32)
    ii = lax.broadcasted_iota(jnp.int32, (S, S), 0)
    jj = lax.broadcasted_iota(jnp.int32, (S, S), 1)

    vcol = _tn(v, jnp.ones((1, 1), f32))
    vi = jnp.broadcast_to(vcol, (S, S))
    vj = jnp.broadcast_to(v, (S, S))
    gt = jnp.sum((vj > vi).astype(f32), axis=1, keepdims=True)
    eqb = jnp.sum(((vj == vi) & (jj < ii)).astype(f32), axis=1,
                  keepdims=True)
    rank = gt + eqb
    selc = rank < float(DOM)

    pp = lax.broadcasted_iota(jnp.int32, (S, 128), 1).astype(f32)
    rank_b = jnp.broadcast_to(rank, (S, 128))
    pt = ((rank_b == pp - 1.0) & jnp.broadcast_to(selc, (S, 128))
          ).astype(f32)
    tok = lax.broadcasted_iota(jnp.int32, (1, S), 1).astype(f32)
    idx_row = lax.dot_general(tok, pt, (((1,), (0,)), ((), ())),
                              precision=_HIGH)
    idx_ref[0, 0:1, :] = idx_row.astype(jnp.int32)

    iota_col0 = lax.broadcasted_iota(jnp.int32, (S, 1), 0) == 0
    sel_col = (selc | iota_col0).astype(f32)
    sel_row = _tn(sel_col, ident_s)
    lt = (ii < jj).astype(f32)
    dom_rank = lax.dot_general(sel_row, lt, (((1,), (0,)), ((), ())),
                               precision=_HIGH)
    unsel_row = 1.0 - sel_row
    f_rank = lax.dot_general(unsel_row, lt, (((1,), (0,)), ((), ())),
                             precision=_HIGH)

    t_id = jnp.floor((f_rank + 0.5) * (1.0 / STEP))
    is_target = ((unsel_row > 0.5) & (f_rank == t_id * float(STEP))
                 & (f_rank < float(STEP * CTX)))
    is_merge = (unsel_row > 0.5) & (~is_target)

    met = met_ref[0]
    ss = jnp.sum(met * met, axis=1, keepdims=True)
    mn = met / jnp.sqrt(ss)
    tt16 = lax.broadcasted_iota(jnp.int32, (16, S), 0).astype(f32)
    wt = ((jnp.broadcast_to(t_id, (16, S)) == tt16)
          & jnp.broadcast_to(is_target, (16, S))).astype(f32)
    tmat = lax.dot_general(wt, mn, (((1,), (0,)), ((), ())),
                           precision=_HIGH)
    sim = lax.dot_general(mn, tmat, (((1,), (1,)), ((), ())),
                          precision=lax.Precision.DEFAULT)
    tcol = lax.broadcasted_iota(jnp.int32, (S, 16), 1).astype(f32)
    sim = jnp.where(tcol < float(CTX), sim, -jnp.inf)
    mx = jnp.max(sim, axis=1, keepdims=True)
    assign_col = jnp.min(jnp.where(sim == mx, tcol, 1e9), axis=1,
                         keepdims=True)
    amat = (jnp.broadcast_to(assign_col, (S, 16)) == tcol).astype(f32)
    counts = lax.dot_general(is_merge.astype(f32), amat,
                             (((1,), (0,)), ((), ())),
                             precision=_HIGH)
    counts = jnp.maximum(counts, 1.0)
    inv_counts_col = _tn(1.0 / counts, jnp.ones((1, 1), f32))
    assign_row = _tn(assign_col, ident_s)
    wm = ((jnp.broadcast_to(assign_row, (16, S)) == tt16)
          & jnp.broadcast_to(is_merge, (16, S))).astype(f32)
    w = wt + wm * inv_counts_col

    rr = lax.broadcasted_iota(jnp.int32, (64, S), 0).astype(f32)
    g = ((jnp.broadcast_to(dom_rank, (64, S)) == rr)
         & jnp.broadcast_to(sel_row > 0.5, (64, S))).astype(f32)

    hid = hid_ref[0]
    out_dom = lax.dot_general(g, hid, (((1,), (0,)), ((), ())),
                              precision=_HIGH)
    out_ctx = lax.dot_general(w, hid, (((1,), (0,)), ((), ())),
                              precision=_HIGH)
    out_ref[0, 0:NSEL, :] = out_dom[0:NSEL, :]
    out_ref[0, NSEL:OUT_T, :] = out_ctx[0:CTX, :]


@jax.jit
def kernel(attn_weights, hidden_states, metric):
    attn_cls = attn_weights[:, :, 0, :]
    out, idx = pl.pallas_call(
        _kernel,
        grid=(B,),
        in_specs=[
            pl.BlockSpec((1, H, S), lambda b: (b, 0, 0)),
            pl.BlockSpec((1, S, D), lambda b: (b, 0, 0)),
            pl.BlockSpec((1, S, DM), lambda b: (b, 0, 0)),
        ],
        out_specs=[
            pl.BlockSpec((1, OUT_T, D), lambda b: (b, 0, 0)),
            pl.BlockSpec((1, 8, 128), lambda b: (b, 0, 0)),
        ],
        out_shape=[
            jax.ShapeDtypeStruct((B, OUT_T, D), jnp.float32),
            jax.ShapeDtypeStruct((B, 8, 128), jnp.int32),
        ],
    )(attn_cls, hidden_states, metric)
    return out, idx[:, 0, :NSEL]


# speedup vs baseline: 2.9663x; 1.4518x over previous
import jax
import jax.numpy as jnp
from jax import lax
from jax.experimental import pallas as pl

B, H, S, D, DM = 8, 16, 577, 1024, 64
DOM = 54
CTX = 10
KEEP = S - (DOM + 1)
STEP = max(1, KEEP // CTX)
NSEL = DOM + 1
OUT_T = NSEL + CTX
CROWS = 80

_HIGH = lax.Precision.HIGHEST
_DEF = lax.Precision.DEFAULT


def _mm(a, bm, prec=_DEF):
    return lax.dot_general(a, bm, (((1,), (0,)), ((), ())), precision=prec)


def _mm_nt(a, bm, prec=_DEF):
    return lax.dot_general(a, bm, (((1,), (1,)), ((), ())), precision=prec)


def _tn(x, ident, prec=_DEF):
    return lax.dot_general(x, ident, (((0,), (0,)), ((), ())),
                           precision=prec)


def _kernel(attn_cls_ref, hid_ref, met_ref, out_ref, idx_ref):
    f32 = jnp.float32
    attn = attn_cls_ref[0]
    v = jnp.sum(attn, axis=0, keepdims=True)
    col0 = lax.broadcasted_iota(jnp.int32, (1, S), 1) == 0
    v = jnp.where(col0, -jnp.inf, v)

    ident_s = (lax.broadcasted_iota(jnp.int32, (S, S), 0)
               == lax.broadcasted_iota(jnp.int32, (S, S), 1)).astype(f32)
    ii = lax.broadcasted_iota(jnp.int32, (S, S), 0)
    jj = lax.broadcasted_iota(jnp.int32, (S, S), 1)

    vcol = _tn(v, jnp.ones((1, 1), f32), _HIGH)
    vi = jnp.broadcast_to(vcol, (S, S))
    vj = jnp.broadcast_to(v, (S, S))
    beats = (vj > vi) | ((vj == vi) & (jj < ii))
    rank = jnp.sum(beats.astype(f32), axis=1, keepdims=True)
    selc = rank < float(DOM)

    pp = lax.broadcasted_iota(jnp.int32, (S, 128), 1).astype(f32)
    pt = ((jnp.broadcast_to(rank, (S, 128)) == pp - 1.0)
          & jnp.broadcast_to(selc, (S, 128))).astype(f32)
    tok = lax.broadcasted_iota(jnp.int32, (1, S), 1).astype(f32)
    idx_row = _mm(tok, pt)
    idx_ref[0, 0:1, :] = idx_row.astype(jnp.int32)

    iota_col0 = lax.broadcasted_iota(jnp.int32, (S, 1), 0) == 0
    sel_col = (selc | iota_col0).astype(f32)
    sel_row = _tn(sel_col, ident_s)
    lt = (ii < jj).astype(f32)
    dom_rank = _mm(sel_row, lt)
    unsel_row = 1.0 - sel_row
    f_rank = tok - dom_rank

    t_id = jnp.floor((f_rank + 0.5) * (1.0 / STEP))
    is_target = ((unsel_row > 0.5) & (f_rank == t_id * float(STEP))
                 & (f_rank < float(STEP * CTX)))
    is_merge = (unsel_row > 0.5) & (~is_target)

    met = met_ref[0]
    ss = jnp.sum(met * met, axis=1, keepdims=True)
    mn = met / jnp.sqrt(ss)
    tt16 = lax.broadcasted_iota(jnp.int32, (16, S), 0).astype(f32)
    wt = ((jnp.broadcast_to(t_id, (16, S)) == tt16)
          & jnp.broadcast_to(is_target, (16, S))).astype(f32)
    tmat = _mm(wt, mn)
    sim = _mm_nt(mn, tmat)
    tcol = lax.broadcasted_iota(jnp.int32, (S, 16), 1).astype(f32)
    sim = jnp.where(tcol < float(CTX), sim, -jnp.inf)
    mx = jnp.max(sim, axis=1, keepdims=True)
    assign_col = jnp.min(jnp.where(sim == mx, tcol, 1e9), axis=1,
                         keepdims=True)
    amat = (jnp.broadcast_to(assign_col, (S, 16)) == tcol).astype(f32)
    counts = _mm(is_merge.astype(f32), amat)
    inv_counts = 1.0 / jnp.maximum(counts, 1.0)
    invc_row = _mm_nt(inv_counts, amat)
    assign_row = _tn(assign_col, ident_s)

    rr = lax.broadcasted_iota(jnp.int32, (CROWS, S), 0).astype(f32)
    dom_part = ((jnp.broadcast_to(dom_rank, (CROWS, S)) == rr)
                & jnp.broadcast_to(sel_row > 0.5, (CROWS, S)))
    tgt_part = ((jnp.broadcast_to(t_id, (CROWS, S)) == rr - 64.0)
                & jnp.broadcast_to(is_target, (CROWS, S)))
    mrg_part = ((jnp.broadcast_to(assign_row, (CROWS, S)) == rr - 64.0)
                & jnp.broadcast_to(is_merge, (CROWS, S)))
    c = (dom_part.astype(f32) + tgt_part.astype(f32)
         + mrg_part.astype(f32) * jnp.broadcast_to(invc_row, (CROWS, S)))

    hid = hid_ref[0]
    out = _mm(c, hid)
    out_ref[0, 0:NSEL, :] = out[0:NSEL, :]
    out_ref[0, NSEL:OUT_T, :] = out[64:64 + CTX, :]


@jax.jit
def kernel(attn_weights, hidden_states, metric):
    attn_cls = attn_weights[:, :, 0, :]
    out, idx = pl.pallas_call(
        _kernel,
        grid=(B,),
        in_specs=[
            pl.BlockSpec((1, H, S), lambda b: (b, 0, 0)),
            pl.BlockSpec((1, S, D), lambda b: (b, 0, 0)),
            pl.BlockSpec((1, S, DM), lambda b: (b, 0, 0)),
        ],
        out_specs=[
            pl.BlockSpec((1, OUT_T, D), lambda b: (b, 0, 0)),
            pl.BlockSpec((1, 8, 128), lambda b: (b, 0, 0)),
        ],
        out_shape=[
            jax.ShapeDtypeStruct((B, OUT_T, D), jnp.float32),
            jax.ShapeDtypeStruct((B, 8, 128), jnp.int32),
        ],
    )(attn_cls, hidden_states, metric)
    return out, idx[:, 0, :NSEL]
